# baseline (speedup 1.0000x reference)
import numpy as np

import jax
import jax.numpy as jnp
from jax.experimental import pallas as pl
from jax.sharding import Mesh, PartitionSpec as P

try:
    from jax import shard_map as _shard_map
except ImportError:
    from jax.experimental.shard_map import shard_map as _shard_map

HIDDEN = 768
NUM_EMB = 32
VOCAB = 8192
V_BLK = 1024


def _kern(ids_ref, emb_ref, pw_ref, out_ref):
    table = jax.lax.dot_general(
        emb_ref[...], pw_ref[...],
        (((1,), (1,)), ((), ())),
        preferred_element_type=jnp.float32,
    )
    ids = ids_ref[0, :]
    s = ids.shape[0]
    onehot = (ids[:, None] == jax.lax.broadcasted_iota(jnp.int32, (s, NUM_EMB), 1)
              ).astype(jnp.bfloat16)
    out_ref[0, :, :] = jnp.dot(onehot, table.astype(jnp.bfloat16),
                               preferred_element_type=jnp.float32)


def _one_hot_call(input_ids, embedding, proj_w):
    b, s = input_ids.shape
    v = proj_w.shape[0]
    return pl.pallas_call(
        _kern,
        grid=(v // V_BLK,),
        in_specs=[
            pl.BlockSpec((b, s), lambda j: (0, 0)),
            pl.BlockSpec((NUM_EMB, HIDDEN), lambda j: (0, 0)),
            pl.BlockSpec((V_BLK, HIDDEN), lambda j: (j, 0)),
        ],
        out_specs=pl.BlockSpec((b, s, V_BLK), lambda j: (0, 0, j)),
        out_shape=jax.ShapeDtypeStruct((b, s, v), jnp.float32),
    )(input_ids, embedding, proj_w)


def kernel(input_ids, embedding, proj_w):
    n_dev = len(jax.devices())
    if n_dev < 2:
        return _one_hot_call(input_ids, embedding, proj_w)
    mesh = Mesh(np.asarray(jax.devices()[:2]), ("x",))
    f = _shard_map(
        _one_hot_call,
        mesh=mesh,
        in_specs=(P(None, None), P(None, None), P("x", None)),
        out_specs=P(None, None, "x"),
        check_vma=False,
    )
    return f(input_ids, embedding, proj_w)


# speedup vs baseline: 7.4997x; 7.4997x over previous
import jax
import jax.numpy as jnp
from jax.experimental import pallas as pl

HIDDEN = 768
NUM_EMB = 32
VOCAB = 8192
V_BLK = 1024
S_BLK = 256


def _table_kern(emb_ref, pw_ref, out_ref):
    out_ref[...] = jax.lax.dot_general(
        emb_ref[...], pw_ref[...],
        (((1,), (1,)), ((), ())),
        preferred_element_type=jnp.float32,
    ).astype(jnp.bfloat16)


def _make_table(embedding, proj_w):
    return pl.pallas_call(
        _table_kern,
        grid=(VOCAB // V_BLK,),
        in_specs=[
            pl.BlockSpec((NUM_EMB, HIDDEN), lambda j: (0, 0)),
            pl.BlockSpec((V_BLK, HIDDEN), lambda j: (j, 0)),
        ],
        out_specs=pl.BlockSpec((NUM_EMB, V_BLK), lambda j: (0, j)),
        out_shape=jax.ShapeDtypeStruct((NUM_EMB, VOCAB), jnp.bfloat16),
    )(embedding, proj_w)


def _gather_kern(ids_ref, table_ref, out_ref):
    ids = ids_ref[0, :]
    s = ids.shape[0]
    onehot = (ids[:, None] == jax.lax.broadcasted_iota(jnp.int32, (s, NUM_EMB), 1)
              ).astype(jnp.bfloat16)
    out_ref[0, :, :] = jnp.dot(onehot, table_ref[...],
                               preferred_element_type=jnp.float32)


def kernel(input_ids, embedding, proj_w):
    b, s = input_ids.shape
    table = _make_table(embedding, proj_w)
    return pl.pallas_call(
        _gather_kern,
        grid=(s // S_BLK,),
        in_specs=[
            pl.BlockSpec((b, S_BLK), lambda i: (0, i)),
            pl.BlockSpec((NUM_EMB, VOCAB), lambda i: (0, 0)),
        ],
        out_specs=pl.BlockSpec((b, S_BLK, VOCAB), lambda i: (0, i, 0)),
        out_shape=jax.ShapeDtypeStruct((b, s, VOCAB), jnp.float32),
    )(input_ids, table)
